# baseline (speedup 1.0000x reference)
import jax
import jax.numpy as jnp
from jax.experimental import pallas as pl

_B = 512
_F = 16384
_Q = 256
_SB = 32


def _rpq_kernel(x_ref, w_ref, c_ref, out_ref):
    x = x_ref[...]
    w = w_ref[...]
    t = jax.lax.dot_general(
        x, w, (((1,), (1,)), ((), ())),
        preferred_element_type=jnp.float32,
    )

    mu = jnp.mean(t, axis=1, keepdims=True)
    var = jnp.mean((t - mu) ** 2, axis=1, keepdims=True)
    tn = (t - mu) / jnp.sqrt(var + 1e-5)

    c = c_ref[...]
    cmu = jnp.mean(c)
    cvar = jnp.mean((c - cmu) ** 2)
    cn = (c - cmu) / jnp.sqrt(cvar + 1e-5)

    d = tn[:, None, :] - cn[None, :, :]
    out_ref[...] = jnp.argmin(d, axis=-1).astype(jnp.int32)


def kernel(input_values, W, codebook):
    csub = codebook[:_Q, :]
    grid = (_B // _SB,)
    return pl.pallas_call(
        _rpq_kernel,
        grid=grid,
        in_specs=[
            pl.BlockSpec((_SB, _F), lambda i: (i, 0)),
            pl.BlockSpec((_Q, _F), lambda i: (0, 0)),
            pl.BlockSpec((_Q, _Q), lambda i: (0, 0)),
        ],
        out_specs=pl.BlockSpec((_SB, _Q), lambda i: (i, 0)),
        out_shape=jax.ShapeDtypeStruct((_B, _Q), jnp.int32),
    )(input_values, W, csub)


# speedup vs baseline: 1.5267x; 1.5267x over previous
import jax
import jax.numpy as jnp
from jax.experimental import pallas as pl

_B = 512
_F = 16384
_Q = 256
_SB = 32


def _rpq_kernel(x_ref, w_ref, ct_ref, out_ref):
    x = x_ref[...]
    w = w_ref[...]
    t = jax.lax.dot_general(
        x, w, (((1,), (1,)), ((), ())),
        preferred_element_type=jnp.float32,
    )

    mu = jnp.mean(t, axis=1, keepdims=True)
    var = jnp.mean((t - mu) ** 2, axis=1, keepdims=True)
    tn = (t - mu) / jnp.sqrt(var + 1e-5)

    ct = ct_ref[...]
    cmu = jnp.mean(ct)
    cvar = jnp.mean((ct - cmu) ** 2)
    cnt = (ct - cmu) / jnp.sqrt(cvar + 1e-5)

    m = jnp.full((_SB, _Q), jnp.inf, jnp.float32)
    idx = jnp.zeros((_SB, _Q), jnp.int32)
    for j in range(_Q):
        d = (jnp.broadcast_to(tn[:, j:j + 1], (_SB, _Q))
             - jnp.broadcast_to(cnt[j:j + 1, :], (_SB, _Q)))
        mask = d < m
        m = jnp.minimum(m, d)
        idx = jnp.where(mask, j, idx)
    out_ref[...] = idx


def kernel(input_values, W, codebook):
    csub_t = codebook[:_Q, :].T
    grid = (_B // _SB,)
    return pl.pallas_call(
        _rpq_kernel,
        grid=grid,
        in_specs=[
            pl.BlockSpec((_SB, _F), lambda i: (i, 0)),
            pl.BlockSpec((_Q, _F), lambda i: (0, 0)),
            pl.BlockSpec((_Q, _Q), lambda i: (0, 0)),
        ],
        out_specs=pl.BlockSpec((_SB, _Q), lambda i: (i, 0)),
        out_shape=jax.ShapeDtypeStruct((_B, _Q), jnp.int32),
    )(input_values, W, csub_t)
